# Initial kernel scaffold; baseline (speedup 1.0000x reference)
import jax
import jax.numpy as jnp
from jax.experimental import pallas as pl


def kernel(indices, table):
    raise NotImplementedError("write your pallas kernel here")


# speedup vs baseline: 1.6924x; 1.6924x over previous
import jax
import jax.numpy as jnp
from jax import lax
from jax.experimental import pallas as pl
from jax.experimental.pallas import tpu as pltpu
from jax.experimental.pallas import tpu_sc as plsc

_NC = 2
_NS = 16
_NW = _NC * _NS
_CH = 256


def _make_body(total, vocab, dim):
    per_w = total // _NW
    nchunks = per_w // _CH
    nfull = dim // 16
    tail = dim - nfull * 16
    toff = dim - 16

    def body(idx_hbm, tab_hbm, out_hbm, tab_v, idx_v, out_v, sem):
        wid = lax.axis_index("s") * _NC + lax.axis_index("c")
        base = wid * per_w
        pltpu.sync_copy(tab_hbm, tab_v)

        def do_chunk(k, carry):
            off = base + k * _CH
            pltpu.sync_copy(idx_hbm.at[pl.ds(off, _CH)], idx_v)

            def group(g, c):
                idx16 = idx_v[pl.ds(g * 16, 16)]
                gbase = g * (16 * dim)
                for l in range(16):
                    src = idx16[l] * dim
                    dst = gbase + l * dim
                    for t in range(nfull):
                        out_v[pl.ds(dst + 16 * t, 16)] = (
                            tab_v[pl.ds(src + 16 * t, 16)])
                    if tail:
                        out_v[pl.ds(dst + toff, 16)] = (
                            tab_v[pl.ds(src + toff, 16)])
                return c

            lax.fori_loop(0, _CH // 16, group, 0)
            pltpu.sync_copy(out_v, out_hbm.at[pl.ds(off * dim, _CH * dim)])
            return carry

        lax.fori_loop(0, nchunks, do_chunk, 0)

    return body


def kernel(indices, table):
    b0, b1 = indices.shape
    vocab, dim = table.shape
    total = b0 * b1
    assert total % (_NW * _CH) == 0 and dim >= 16
    idx1d = indices.reshape(total).astype(jnp.int32)
    tab1d = table.reshape(vocab * dim)

    run = pl.kernel(
        _make_body(total, vocab, dim),
        out_type=jax.ShapeDtypeStruct((total * dim,), jnp.float32),
        mesh=plsc.VectorSubcoreMesh(core_axis_name="c", subcore_axis_name="s"),
        scratch_types=[
            pltpu.VMEM((vocab * dim,), jnp.float32),
            pltpu.VMEM((_CH,), jnp.int32),
            pltpu.VMEM((_CH * dim,), jnp.float32),
            pltpu.SemaphoreType.DMA,
        ],
    )
    return run(idx1d, tab1d).reshape(b0, b1, dim)
